# baseline (speedup 1.0000x reference)
import functools

import jax
import jax.numpy as jnp
from jax import lax
from jax.experimental import pallas as pl
from jax.experimental.pallas import tpu as pltpu
from jax.experimental.pallas import tpu_sc as plsc

N = 10000
E = 320000
F = 128

NC = 2
NS = 16
NW = NC * NS
EPW = E // NW
CHUNK = 80
NCHUNK = EPW // CHUNK
BLK = 25
NBLK = NCHUNK // BLK
SLAB = 624
TAIL = N - NS * SLAB
DEN = 10240


def _tc_in_body(x_ref, w_ref, asrc_ref, adst_ref, h_ref, as_ref, ad_ref):
    h = jnp.dot(x_ref[...], w_ref[...], preferred_element_type=jnp.float32)
    h_ref[...] = h
    as_ref[...] = jnp.sum(h * asrc_ref[...], axis=1, keepdims=True)
    ad_ref[...] = jnp.sum(h * adst_ref[...], axis=1, keepdims=True)


_tc_in = pl.pallas_call(
    _tc_in_body,
    out_shape=(
        jax.ShapeDtypeStruct((N, F), jnp.float32),
        jax.ShapeDtypeStruct((N, 1), jnp.float32),
        jax.ShapeDtypeStruct((N, 1), jnp.float32),
    ),
)


def _normalize(pnum_ref, pden_ref, b_ref, g_ref, be_ref):
    num = pnum_ref[0] + pnum_ref[1]
    den = pden_ref[0] + pden_ref[1]
    y = num / (den + 1e-16) + b_ref[...]
    y = jnp.maximum(y, 0.0)
    m = jnp.mean(y, axis=0, keepdims=True)
    v = jnp.mean((y - m) * (y - m), axis=0, keepdims=True)
    return g_ref[...] * (y - m) * lax.rsqrt(v + 1e-5) + be_ref[...]


def _tc_mid_body(pnum_ref, pden_ref, b_ref, g_ref, be_ref, w_ref, asrc_ref,
                 adst_ref, h_ref, as_ref, ad_ref):
    y = _normalize(pnum_ref, pden_ref, b_ref, g_ref, be_ref)
    h = jnp.dot(y, w_ref[...], preferred_element_type=jnp.float32)
    h_ref[...] = h
    as_ref[...] = jnp.sum(h * asrc_ref[...], axis=1, keepdims=True)
    ad_ref[...] = jnp.sum(h * adst_ref[...], axis=1, keepdims=True)


_tc_mid = pl.pallas_call(
    _tc_mid_body,
    out_shape=(
        jax.ShapeDtypeStruct((N, F), jnp.float32),
        jax.ShapeDtypeStruct((N, 1), jnp.float32),
        jax.ShapeDtypeStruct((N, 1), jnp.float32),
    ),
)


def _tc_out_body(pnum_ref, pden_ref, b_ref, g_ref, be_ref, o_ref):
    o_ref[...] = _normalize(pnum_ref, pden_ref, b_ref, g_ref, be_ref)


_tc_out = pl.pallas_call(
    _tc_out_body,
    out_shape=jax.ShapeDtypeStruct((N, F), jnp.float32),
)


@functools.cache
def _make_sc_edge():
  mesh = plsc.VectorSubcoreMesh(core_axis_name="c", subcore_axis_name="s",
                                num_cores=NC, num_subcores=NS)

  @functools.partial(
      pl.kernel,
      out_type=(
          jax.ShapeDtypeStruct((NC, N, F), jnp.float32),
          jax.ShapeDtypeStruct((NC, DEN), jnp.float32),
      ),
      mesh=mesh,
      scratch_types=(
          pltpu.VMEM((2, BLK, CHUNK), jnp.int32),
          pltpu.VMEM((2, BLK, CHUNK), jnp.int32),
          pltpu.VMEM((2, CHUNK, F), jnp.float32),
          pltpu.VMEM((N,), jnp.float32),
          pltpu.VMEM((N,), jnp.float32),
          pltpu.VMEM((2, CHUNK), jnp.float32),
          pltpu.VMEM_SHARED((N, F), jnp.float32),
          pltpu.VMEM_SHARED((DEN,), jnp.float32),
          pltpu.SemaphoreType.DMA((2,)),
          pltpu.SemaphoreType.DMA((2,)),
          pltpu.SemaphoreType.DMA,
      ),
      compiler_params=pltpu.CompilerParams(needs_layout_passes=False,
                                           use_tc_tiling_on_sc=False),
  )
  def _sc_edge(h_hbm, src_hbm, dst_hbm, as_hbm, ad_hbm, pnum_hbm, pden_hbm,
               sidxb, didxb, rows, asv, adv, eeb, num_sh, den_sh,
               gsem, ssem, isem):
    c = lax.axis_index("c")
    s = lax.axis_index("s")
    wid = s * NC + c
    zero16 = jnp.zeros((16,), jnp.float32)

    pltpu.sync_copy(as_hbm, asv)
    pltpu.sync_copy(ad_hbm, adv)
    pltpu.sync_copy(src_hbm.at[wid, pl.ds(0, BLK)], sidxb.at[0])
    pltpu.sync_copy(dst_hbm.at[wid, pl.ds(0, BLK)], didxb.at[0])

    def _zrow(i, _):
      for k in range(F // 16):
        rows[0, i, pl.ds(k * 16, 16)] = zero16
      return 0

    lax.fori_loop(0, CHUNK, _zrow, 0)
    for g in range(CHUNK // 16):
      eeb[0, pl.ds(g * 16, 16)] = zero16

    for t in range(7):
      pltpu.sync_copy(rows.at[0],
                      num_sh.at[pl.ds(s * SLAB + t * CHUNK, CHUNK)])
    pltpu.sync_copy(rows.at[0, pl.ds(0, SLAB - 7 * CHUNK)],
                    num_sh.at[pl.ds(s * SLAB + 7 * CHUNK, SLAB - 7 * CHUNK)])

    @pl.when(s == NS - 1)
    def _ztail():
      pltpu.sync_copy(rows.at[0, pl.ds(0, TAIL)],
                      num_sh.at[pl.ds(NS * SLAB, TAIL)])
    for t in range(DEN // NS // CHUNK):
      pltpu.sync_copy(eeb.at[0],
                      den_sh.at[pl.ds(s * (DEN // NS) + t * CHUNK, CHUNK)])

    def _gather(pb, pos, p):
      pltpu.async_copy(h_hbm.at[sidxb.at[pb, pos]], rows.at[p], gsem.at[p])

    def _wait_gather(p):
      pltpu.make_async_copy(h_hbm.at[sidxb.at[0, 0]], rows.at[p],
                            gsem.at[p]).wait()

    def _wait_scatter(p):
      pltpu.make_async_copy(rows.at[p], num_sh.at[didxb.at[0, 0]],
                            ssem.at[p]).wait()
      pltpu.make_async_copy(eeb.at[p], den_sh.at[didxb.at[0, 0]],
                            ssem.at[p]).wait()

    plsc.subcore_barrier()
    _gather(0, 0, 0)

    def _chunk(j, _):
      p = j & 1
      q = 1 - p
      blk = j // BLK
      pos = j - blk * BLK
      pb = blk & 1

      @pl.when(jnp.logical_and(pos == 0, blk < NBLK - 1))
      def _iprefetch():
        pltpu.async_copy(src_hbm.at[wid, pl.ds((blk + 1) * BLK, BLK)],
                         sidxb.at[1 - pb], isem)
        pltpu.async_copy(dst_hbm.at[wid, pl.ds((blk + 1) * BLK, BLK)],
                         didxb.at[1 - pb], isem)

      @pl.when(jnp.logical_and(pos == BLK - 1, blk < NBLK - 1))
      def _iwait():
        pltpu.make_async_copy(src_hbm.at[wid, pl.ds(0, BLK)], sidxb.at[0],
                              isem).wait()
        pltpu.make_async_copy(dst_hbm.at[wid, pl.ds(0, BLK)], didxb.at[0],
                              isem).wait()

      @pl.when(j >= 1)
      def _recycle():
        _wait_scatter(q)

      @pl.when(j + 1 < NCHUNK)
      def _prefetch():
        nj = j + 1
        nblk = nj // BLK
        _gather(nblk & 1, nj - nblk * BLK, q)

      for g in range(CHUNK // 16):
        sidx = sidxb[pb, pos, pl.ds(g * 16, 16)]
        didx = didxb[pb, pos, pl.ds(g * 16, 16)]
        e = plsc.load_gather(asv, [sidx]) + plsc.load_gather(adv, [didx])
        e = jnp.where(e >= 0.0, e, e * 0.2)
        eeb[p, pl.ds(g * 16, 16)] = jnp.exp(e)

      _wait_gather(p)

      def _scale(g, _):
        ee = eeb[p, pl.ds(g * 16, 16)]
        for l in range(16):
          w = ee[l]
          ri = g * 16 + l
          for k in range(F // 16):
            rows[p, ri, pl.ds(k * 16, 16)] = rows[p, ri, pl.ds(k * 16, 16)] * w
        return 0

      lax.fori_loop(0, CHUNK // 16, _scale, 0)

      pltpu.async_copy(rows.at[p], num_sh.at[didxb.at[pb, pos]], ssem.at[p],
                       add=True)
      pltpu.async_copy(eeb.at[p], den_sh.at[didxb.at[pb, pos]], ssem.at[p],
                       add=True)
      return 0

    lax.fori_loop(0, NCHUNK, _chunk, 0)
    _wait_scatter((NCHUNK - 1) & 1)
    plsc.subcore_barrier()

    pltpu.sync_copy(num_sh.at[pl.ds(s * SLAB, SLAB)],
                    pnum_hbm.at[c, pl.ds(s * SLAB, SLAB)])

    @pl.when(s == NS - 1)
    def _wtail():
      pltpu.sync_copy(num_sh.at[pl.ds(NS * SLAB, TAIL)],
                      pnum_hbm.at[c, pl.ds(NS * SLAB, TAIL)])
    pltpu.sync_copy(den_sh.at[pl.ds(s * (DEN // NS), DEN // NS)],
                    pden_hbm.at[c, pl.ds(s * (DEN // NS), DEN // NS)])

  return _sc_edge


def _layer_edge(h, a_s, a_d, src3, dst3):
    pnum, pden = _make_sc_edge()(h, src3, dst3,
                                 a_s.reshape(N), a_d.reshape(N))
    pden = pden[:, :N].reshape(NC, N, 1)
    return pnum, pden


def kernel(x, edge_index, W1, att_src1, att_dst1, b1, gamma1, beta1,
           W2, att_src2, att_dst2, b2, gamma2, beta2):
    ei = edge_index.astype(jnp.int32)
    src3 = ei[0].reshape(NW, NBLK * BLK, CHUNK)
    dst3 = ei[1].reshape(NW, NBLK * BLK, CHUNK)
    r = lambda a: a.reshape(1, F)

    h, a_s, a_d = _tc_in(x, W1, r(att_src1), r(att_dst1))
    pnum, pden = _layer_edge(h, a_s, a_d, src3, dst3)
    h, a_s, a_d = _tc_mid(pnum, pden, r(b1), r(gamma1), r(beta1),
                          W2, r(att_src2), r(att_dst2))
    pnum, pden = _layer_edge(h, a_s, a_d, src3, dst3)
    return _tc_out(pnum, pden, r(b2), r(gamma2), r(beta2))


# speedup vs baseline: 2.5518x; 2.5518x over previous
import functools

import jax
import jax.numpy as jnp
from jax import lax
from jax.experimental import pallas as pl
from jax.experimental.pallas import tpu as pltpu
from jax.experimental.pallas import tpu_sc as plsc

N = 10000
E = 320000
F = 128

NC = 2
NS = 16
NW = NC * NS
EPW = E // NW
CHUNK = 80
NCHUNK = EPW // CHUNK
BLK = 25
NBLK = NCHUNK // BLK
SLAB = 624
TAIL = N - NS * SLAB
DEN = 10240


def _tc_in_body(x_ref, w_ref, asrc_ref, adst_ref, h_ref, as_ref, ad_ref):
    h = jnp.dot(x_ref[...], w_ref[...], preferred_element_type=jnp.float32)
    h_ref[...] = h
    as_ref[...] = jnp.sum(h * asrc_ref[...], axis=1, keepdims=True)
    ad_ref[...] = jnp.sum(h * adst_ref[...], axis=1, keepdims=True)


_tc_in = pl.pallas_call(
    _tc_in_body,
    out_shape=(
        jax.ShapeDtypeStruct((N, F), jnp.float32),
        jax.ShapeDtypeStruct((N, 1), jnp.float32),
        jax.ShapeDtypeStruct((N, 1), jnp.float32),
    ),
)


def _normalize(pnum_ref, pden_ref, b_ref, g_ref, be_ref):
    num = pnum_ref[0] + pnum_ref[1]
    den = pden_ref[0] + pden_ref[1]
    y = num / (den + 1e-16) + b_ref[...]
    y = jnp.maximum(y, 0.0)
    m = jnp.mean(y, axis=0, keepdims=True)
    v = jnp.mean((y - m) * (y - m), axis=0, keepdims=True)
    return g_ref[...] * (y - m) * lax.rsqrt(v + 1e-5) + be_ref[...]


def _tc_mid_body(pnum_ref, pden_ref, b_ref, g_ref, be_ref, w_ref, asrc_ref,
                 adst_ref, h_ref, as_ref, ad_ref):
    y = _normalize(pnum_ref, pden_ref, b_ref, g_ref, be_ref)
    h = jnp.dot(y, w_ref[...], preferred_element_type=jnp.float32)
    h_ref[...] = h
    as_ref[...] = jnp.sum(h * asrc_ref[...], axis=1, keepdims=True)
    ad_ref[...] = jnp.sum(h * adst_ref[...], axis=1, keepdims=True)


_tc_mid = pl.pallas_call(
    _tc_mid_body,
    out_shape=(
        jax.ShapeDtypeStruct((N, F), jnp.float32),
        jax.ShapeDtypeStruct((N, 1), jnp.float32),
        jax.ShapeDtypeStruct((N, 1), jnp.float32),
    ),
)


def _tc_out_body(pnum_ref, pden_ref, b_ref, g_ref, be_ref, o_ref):
    o_ref[...] = _normalize(pnum_ref, pden_ref, b_ref, g_ref, be_ref)


_tc_out = pl.pallas_call(
    _tc_out_body,
    out_shape=jax.ShapeDtypeStruct((N, F), jnp.float32),
)


@functools.cache
def _make_sc_edge():
  mesh = plsc.VectorSubcoreMesh(core_axis_name="c", subcore_axis_name="s",
                                num_cores=NC, num_subcores=NS)

  @functools.partial(
      pl.kernel,
      out_type=(
          jax.ShapeDtypeStruct((NC, N, F), jnp.float32),
          jax.ShapeDtypeStruct((NC, DEN), jnp.float32),
      ),
      mesh=mesh,
      scratch_types=(
          pltpu.VMEM((2, BLK, CHUNK), jnp.int32),
          pltpu.VMEM((2, BLK, CHUNK), jnp.int32),
          pltpu.VMEM((2, CHUNK, F), jnp.float32),
          pltpu.VMEM((N,), jnp.float32),
          pltpu.VMEM((N,), jnp.float32),
          pltpu.VMEM((2, CHUNK), jnp.float32),
          pltpu.VMEM_SHARED((N, F), jnp.float32),
          pltpu.VMEM_SHARED((DEN,), jnp.float32),
          pltpu.SemaphoreType.DMA((2,)),
          pltpu.SemaphoreType.DMA((2,)),
          pltpu.SemaphoreType.DMA,
      ),
      compiler_params=pltpu.CompilerParams(needs_layout_passes=False,
                                           use_tc_tiling_on_sc=False),
  )
  def _sc_edge(h_hbm, src_hbm, dst_hbm, as_hbm, ad_hbm, pnum_hbm, pden_hbm,
               sidxb, didxb, rows, asv, adv, eeb, num_sh, den_sh,
               gsem, ssem, isem):
    c = lax.axis_index("c")
    s = lax.axis_index("s")
    wid = s * NC + c
    zero16 = jnp.zeros((16,), jnp.float32)

    pltpu.sync_copy(as_hbm, asv)
    pltpu.sync_copy(ad_hbm, adv)
    pltpu.sync_copy(src_hbm.at[wid, pl.ds(0, BLK)], sidxb.at[0])
    pltpu.sync_copy(dst_hbm.at[wid, pl.ds(0, BLK)], didxb.at[0])

    def _zrow(i, _):
      for k in range(F // 16):
        rows[0, i, pl.ds(k * 16, 16)] = zero16
      return 0

    lax.fori_loop(0, CHUNK, _zrow, 0)
    for g in range(CHUNK // 16):
      eeb[0, pl.ds(g * 16, 16)] = zero16

    for t in range(7):
      pltpu.sync_copy(rows.at[0],
                      num_sh.at[pl.ds(s * SLAB + t * CHUNK, CHUNK)])
    pltpu.sync_copy(rows.at[0, pl.ds(0, SLAB - 7 * CHUNK)],
                    num_sh.at[pl.ds(s * SLAB + 7 * CHUNK, SLAB - 7 * CHUNK)])

    @pl.when(s == NS - 1)
    def _ztail():
      pltpu.sync_copy(rows.at[0, pl.ds(0, TAIL)],
                      num_sh.at[pl.ds(NS * SLAB, TAIL)])
    for t in range(DEN // NS // CHUNK):
      pltpu.sync_copy(eeb.at[0],
                      den_sh.at[pl.ds(s * (DEN // NS) + t * CHUNK, CHUNK)])

    def _gather(pb, pos, p):
      pltpu.async_copy(h_hbm.at[sidxb.at[pb, pos]], rows.at[p], gsem.at[p])

    def _wait_gather(p):
      pltpu.make_async_copy(h_hbm.at[sidxb.at[0, 0]], rows.at[p],
                            gsem.at[p]).wait()

    def _wait_scatter(p):
      pltpu.make_async_copy(rows.at[p], num_sh.at[didxb.at[0, 0]],
                            ssem.at[p]).wait()
      pltpu.make_async_copy(eeb.at[p], den_sh.at[didxb.at[0, 0]],
                            ssem.at[p]).wait()

    plsc.subcore_barrier()
    _gather(0, 0, 0)

    def _chunk(j, _):
      p = j & 1
      q = 1 - p
      blk = j // BLK
      pos = j - blk * BLK
      pb = blk & 1

      @pl.when(jnp.logical_and(pos == 0, blk < NBLK - 1))
      def _iprefetch():
        pltpu.async_copy(src_hbm.at[wid, pl.ds((blk + 1) * BLK, BLK)],
                         sidxb.at[1 - pb], isem)
        pltpu.async_copy(dst_hbm.at[wid, pl.ds((blk + 1) * BLK, BLK)],
                         didxb.at[1 - pb], isem)

      @pl.when(jnp.logical_and(pos == BLK - 1, blk < NBLK - 1))
      def _iwait():
        pltpu.make_async_copy(src_hbm.at[wid, pl.ds(0, BLK)], sidxb.at[0],
                              isem).wait()
        pltpu.make_async_copy(dst_hbm.at[wid, pl.ds(0, BLK)], didxb.at[0],
                              isem).wait()

      @pl.when(j >= 1)
      def _recycle():
        _wait_scatter(q)

      @pl.when(j + 1 < NCHUNK)
      def _prefetch():
        nj = j + 1
        nblk = nj // BLK
        _gather(nblk & 1, nj - nblk * BLK, q)

      for g in range(CHUNK // 16):
        sidx = sidxb[pb, pos, pl.ds(g * 16, 16)]
        didx = didxb[pb, pos, pl.ds(g * 16, 16)]
        e = plsc.load_gather(asv, [sidx]) + plsc.load_gather(adv, [didx])
        e = jnp.where(e >= 0.0, e, e * 0.2)
        eeb[p, pl.ds(g * 16, 16)] = jnp.exp(e)

      _wait_gather(p)

      for g in range(CHUNK // 16):
        ee = eeb[p, pl.ds(g * 16, 16)]
        for l in range(16):
          w = ee[l]
          ri = g * 16 + l
          for k in range(F // 16):
            rows[p, ri, pl.ds(k * 16, 16)] = rows[p, ri, pl.ds(k * 16, 16)] * w

      pltpu.async_copy(rows.at[p], num_sh.at[didxb.at[pb, pos]], ssem.at[p],
                       add=True)
      pltpu.async_copy(eeb.at[p], den_sh.at[didxb.at[pb, pos]], ssem.at[p],
                       add=True)
      return 0

    lax.fori_loop(0, NCHUNK, _chunk, 0)
    _wait_scatter((NCHUNK - 1) & 1)
    plsc.subcore_barrier()

    pltpu.sync_copy(num_sh.at[pl.ds(s * SLAB, SLAB)],
                    pnum_hbm.at[c, pl.ds(s * SLAB, SLAB)])

    @pl.when(s == NS - 1)
    def _wtail():
      pltpu.sync_copy(num_sh.at[pl.ds(NS * SLAB, TAIL)],
                      pnum_hbm.at[c, pl.ds(NS * SLAB, TAIL)])
    pltpu.sync_copy(den_sh.at[pl.ds(s * (DEN // NS), DEN // NS)],
                    pden_hbm.at[c, pl.ds(s * (DEN // NS), DEN // NS)])

  return _sc_edge


def _layer_edge(h, a_s, a_d, src3, dst3):
    pnum, pden = _make_sc_edge()(h, src3, dst3,
                                 a_s.reshape(N), a_d.reshape(N))
    pden = pden[:, :N].reshape(NC, N, 1)
    return pnum, pden


def kernel(x, edge_index, W1, att_src1, att_dst1, b1, gamma1, beta1,
           W2, att_src2, att_dst2, b2, gamma2, beta2):
    ei = edge_index.astype(jnp.int32)
    src3 = ei[0].reshape(NW, NBLK * BLK, CHUNK)
    dst3 = ei[1].reshape(NW, NBLK * BLK, CHUNK)
    r = lambda a: a.reshape(1, F)

    h, a_s, a_d = _tc_in(x, W1, r(att_src1), r(att_dst1))
    pnum, pden = _layer_edge(h, a_s, a_d, src3, dst3)
    h, a_s, a_d = _tc_mid(pnum, pden, r(b1), r(gamma1), r(beta1),
                          W2, r(att_src2), r(att_dst2))
    pnum, pden = _layer_edge(h, a_s, a_d, src3, dst3)
    return _tc_out(pnum, pden, r(b2), r(gamma2), r(beta2))


# speedup vs baseline: 2.5611x; 1.0036x over previous
import functools

import jax
import jax.numpy as jnp
from jax import lax
from jax.experimental import pallas as pl
from jax.experimental.pallas import tpu as pltpu
from jax.experimental.pallas import tpu_sc as plsc

N = 10000
E = 320000
F = 128

NC = 2
NS = 16
NW = NC * NS
EPW = E // NW
CHUNK = 80
NCHUNK = EPW // CHUNK
BLK = 25
NBLK = NCHUNK // BLK
SLAB = 624
TAIL = N - NS * SLAB
DEN = 10240


def _tc_in_body(x_ref, w_ref, asrc_ref, adst_ref, h_ref, as_ref, ad_ref):
    h = jnp.dot(x_ref[...], w_ref[...], preferred_element_type=jnp.float32)
    h_ref[...] = h
    as_ref[...] = jnp.sum(h * asrc_ref[...], axis=1, keepdims=True)
    ad_ref[...] = jnp.sum(h * adst_ref[...], axis=1, keepdims=True)


_tc_in = pl.pallas_call(
    _tc_in_body,
    out_shape=(
        jax.ShapeDtypeStruct((N, F), jnp.float32),
        jax.ShapeDtypeStruct((N, 1), jnp.float32),
        jax.ShapeDtypeStruct((N, 1), jnp.float32),
    ),
)


def _normalize(pnum_ref, pden_ref, b_ref, g_ref, be_ref):
    num = pnum_ref[0] + pnum_ref[1]
    den = pden_ref[0] + pden_ref[1]
    y = num / (den + 1e-16) + b_ref[...]
    y = jnp.maximum(y, 0.0)
    m = jnp.mean(y, axis=0, keepdims=True)
    v = jnp.mean((y - m) * (y - m), axis=0, keepdims=True)
    return g_ref[...] * (y - m) * lax.rsqrt(v + 1e-5) + be_ref[...]


def _tc_mid_body(pnum_ref, pden_ref, b_ref, g_ref, be_ref, w_ref, asrc_ref,
                 adst_ref, h_ref, as_ref, ad_ref):
    y = _normalize(pnum_ref, pden_ref, b_ref, g_ref, be_ref)
    h = jnp.dot(y, w_ref[...], preferred_element_type=jnp.float32)
    h_ref[...] = h
    as_ref[...] = jnp.sum(h * asrc_ref[...], axis=1, keepdims=True)
    ad_ref[...] = jnp.sum(h * adst_ref[...], axis=1, keepdims=True)


_tc_mid = pl.pallas_call(
    _tc_mid_body,
    out_shape=(
        jax.ShapeDtypeStruct((N, F), jnp.float32),
        jax.ShapeDtypeStruct((N, 1), jnp.float32),
        jax.ShapeDtypeStruct((N, 1), jnp.float32),
    ),
)


def _tc_out_body(pnum_ref, pden_ref, b_ref, g_ref, be_ref, o_ref):
    o_ref[...] = _normalize(pnum_ref, pden_ref, b_ref, g_ref, be_ref)


_tc_out = pl.pallas_call(
    _tc_out_body,
    out_shape=jax.ShapeDtypeStruct((N, F), jnp.float32),
)


@functools.cache
def _make_sc_edge():
  mesh = plsc.VectorSubcoreMesh(core_axis_name="c", subcore_axis_name="s",
                                num_cores=NC, num_subcores=NS)

  @functools.partial(
      pl.kernel,
      out_type=(
          jax.ShapeDtypeStruct((NC, N, F), jnp.float32),
          jax.ShapeDtypeStruct((NC, DEN), jnp.float32),
      ),
      mesh=mesh,
      scratch_types=(
          pltpu.VMEM((2, BLK, CHUNK), jnp.int32),
          pltpu.VMEM((2, BLK, CHUNK), jnp.int32),
          pltpu.VMEM((2, CHUNK, F), jnp.float32),
          pltpu.VMEM((N,), jnp.float32),
          pltpu.VMEM((N,), jnp.float32),
          pltpu.VMEM((2, CHUNK), jnp.float32),
          pltpu.VMEM_SHARED((N, F), jnp.float32),
          pltpu.VMEM_SHARED((DEN,), jnp.float32),
          pltpu.SemaphoreType.DMA((2,)),
          pltpu.SemaphoreType.DMA((2,)),
          pltpu.SemaphoreType.DMA((2,)),
          pltpu.SemaphoreType.DMA,
      ),
      compiler_params=pltpu.CompilerParams(needs_layout_passes=False,
                                           use_tc_tiling_on_sc=False),
  )
  def _sc_edge(h_hbm, src_hbm, dst_hbm, as_hbm, ad_hbm, pnum_hbm, pden_hbm,
               sidxb, didxb, rows, asv, adv, eeb, num_sh, den_sh,
               gsem, ssem, esem, isem):
    c = lax.axis_index("c")
    s = lax.axis_index("s")
    wid = s * NC + c
    zero16 = jnp.zeros((16,), jnp.float32)

    pltpu.sync_copy(as_hbm, asv)
    pltpu.sync_copy(ad_hbm, adv)
    pltpu.sync_copy(src_hbm.at[wid, pl.ds(0, BLK)], sidxb.at[0])
    pltpu.sync_copy(dst_hbm.at[wid, pl.ds(0, BLK)], didxb.at[0])

    def _zrow(i, _):
      for k in range(F // 16):
        rows[0, i, pl.ds(k * 16, 16)] = zero16
      return 0

    lax.fori_loop(0, CHUNK, _zrow, 0)
    for g in range(CHUNK // 16):
      eeb[0, pl.ds(g * 16, 16)] = zero16

    for t in range(7):
      pltpu.sync_copy(rows.at[0],
                      num_sh.at[pl.ds(s * SLAB + t * CHUNK, CHUNK)])
    pltpu.sync_copy(rows.at[0, pl.ds(0, SLAB - 7 * CHUNK)],
                    num_sh.at[pl.ds(s * SLAB + 7 * CHUNK, SLAB - 7 * CHUNK)])

    @pl.when(s == NS - 1)
    def _ztail():
      pltpu.sync_copy(rows.at[0, pl.ds(0, TAIL)],
                      num_sh.at[pl.ds(NS * SLAB, TAIL)])
    for t in range(DEN // NS // CHUNK):
      pltpu.sync_copy(eeb.at[0],
                      den_sh.at[pl.ds(s * (DEN // NS) + t * CHUNK, CHUNK)])

    def _gather(pb, pos, p):
      pltpu.async_copy(h_hbm.at[sidxb.at[pb, pos]], rows.at[p], gsem.at[p])

    def _wait_gather(p):
      pltpu.make_async_copy(h_hbm.at[sidxb.at[0, 0]], rows.at[p],
                            gsem.at[p]).wait()

    def _wait_rows_scatter(p):
      pltpu.make_async_copy(rows.at[p], num_sh.at[didxb.at[0, 0]],
                            ssem.at[p]).wait()

    def _wait_den_scatter(p):
      pltpu.make_async_copy(eeb.at[p], den_sh.at[didxb.at[0, 0]],
                            esem.at[p]).wait()

    plsc.subcore_barrier()
    _gather(0, 0, 0)

    def _chunk(j, _):
      p = j & 1
      q = 1 - p
      blk = j // BLK
      pos = j - blk * BLK
      pb = blk & 1

      @pl.when(jnp.logical_and(pos == 0, blk < NBLK - 1))
      def _iprefetch():
        pltpu.async_copy(src_hbm.at[wid, pl.ds((blk + 1) * BLK, BLK)],
                         sidxb.at[1 - pb], isem)
        pltpu.async_copy(dst_hbm.at[wid, pl.ds((blk + 1) * BLK, BLK)],
                         didxb.at[1 - pb], isem)

      @pl.when(jnp.logical_and(pos == BLK - 1, blk < NBLK - 1))
      def _iwait():
        pltpu.make_async_copy(src_hbm.at[wid, pl.ds(0, BLK)], sidxb.at[0],
                              isem).wait()
        pltpu.make_async_copy(dst_hbm.at[wid, pl.ds(0, BLK)], didxb.at[0],
                              isem).wait()

      @pl.when(j >= 2)
      def _edrain():
        _wait_den_scatter(p)

      for g in range(CHUNK // 16):
        sidx = sidxb[pb, pos, pl.ds(g * 16, 16)]
        didx = didxb[pb, pos, pl.ds(g * 16, 16)]
        e = plsc.load_gather(asv, [sidx]) + plsc.load_gather(adv, [didx])
        e = jnp.where(e >= 0.0, e, e * 0.2)
        eeb[p, pl.ds(g * 16, 16)] = jnp.exp(e)

      @pl.when(j >= 1)
      def _recycle():
        _wait_rows_scatter(q)

      @pl.when(j + 1 < NCHUNK)
      def _prefetch():
        nj = j + 1
        nblk = nj // BLK
        _gather(nblk & 1, nj - nblk * BLK, q)

      _wait_gather(p)

      for g in range(CHUNK // 16):
        ee = eeb[p, pl.ds(g * 16, 16)]
        for l in range(16):
          w = ee[l]
          ri = g * 16 + l
          for k in range(F // 16):
            rows[p, ri, pl.ds(k * 16, 16)] = rows[p, ri, pl.ds(k * 16, 16)] * w

      pltpu.async_copy(rows.at[p], num_sh.at[didxb.at[pb, pos]], ssem.at[p],
                       add=True)
      pltpu.async_copy(eeb.at[p], den_sh.at[didxb.at[pb, pos]], esem.at[p],
                       add=True)
      return 0

    lax.fori_loop(0, NCHUNK, _chunk, 0)
    _wait_rows_scatter((NCHUNK - 1) & 1)
    _wait_den_scatter(0)
    _wait_den_scatter(1)
    plsc.subcore_barrier()

    pltpu.sync_copy(num_sh.at[pl.ds(s * SLAB, SLAB)],
                    pnum_hbm.at[c, pl.ds(s * SLAB, SLAB)])

    @pl.when(s == NS - 1)
    def _wtail():
      pltpu.sync_copy(num_sh.at[pl.ds(NS * SLAB, TAIL)],
                      pnum_hbm.at[c, pl.ds(NS * SLAB, TAIL)])
    pltpu.sync_copy(den_sh.at[pl.ds(s * (DEN // NS), DEN // NS)],
                    pden_hbm.at[c, pl.ds(s * (DEN // NS), DEN // NS)])

  return _sc_edge


def _layer_edge(h, a_s, a_d, src3, dst3):
    pnum, pden = _make_sc_edge()(h, src3, dst3,
                                 a_s.reshape(N), a_d.reshape(N))
    pden = pden[:, :N].reshape(NC, N, 1)
    return pnum, pden


def kernel(x, edge_index, W1, att_src1, att_dst1, b1, gamma1, beta1,
           W2, att_src2, att_dst2, b2, gamma2, beta2):
    ei = edge_index.astype(jnp.int32)
    src3 = ei[0].reshape(NW, NBLK * BLK, CHUNK)
    dst3 = ei[1].reshape(NW, NBLK * BLK, CHUNK)
    r = lambda a: a.reshape(1, F)

    h, a_s, a_d = _tc_in(x, W1, r(att_src1), r(att_dst1))
    pnum, pden = _layer_edge(h, a_s, a_d, src3, dst3)
    h, a_s, a_d = _tc_mid(pnum, pden, r(b1), r(gamma1), r(beta1),
                          W2, r(att_src2), r(att_dst2))
    pnum, pden = _layer_edge(h, a_s, a_d, src3, dst3)
    return _tc_out(pnum, pden, r(b2), r(gamma2), r(beta2))
